# baseline (speedup 1.0000x reference)
import functools

import jax
import jax.numpy as jnp
from jax.experimental import pallas as pl
from jax.experimental.pallas import tpu as pltpu

D = 8192
D_OUT = 4096
N_FIXED = 8
NSTREAMS = 4
BLK = 128


def _ffm_kernel(xp_ref, attrs_ref, *refs):
    w_refs = refs[:NSTREAMS]
    b_ref, out_ref = refs[NSTREAMS], refs[NSTREAMS + 1]
    xs = pltpu.roll(xp_ref[...], N_FIXED, axis=1)
    col = jax.lax.broadcasted_iota(jnp.int32, (1, D), 1)
    inp = jnp.where(col < N_FIXED, attrs_ref[...], xs)
    for j in range(NSTREAMS):
        acc = jax.lax.dot_general(
            inp, w_refs[j][...], (((1,), (1,)), ((), ())),
            preferred_element_type=jnp.float32)
        out_ref[:, j * BLK:(j + 1) * BLK] = acc + b_ref[:, j * BLK:(j + 1) * BLK]


@functools.partial(jax.jit, static_argnames=())
def kernel(x, attrs_init, W, b):
    xp = jnp.pad(x, ((0, 0), (0, N_FIXED)))
    b2 = b.reshape(1, D_OUT)
    grid = (D_OUT // (BLK * NSTREAMS),)
    w_specs = [
        pl.BlockSpec((BLK, D), functools.partial(
            lambda i, jj: (NSTREAMS * i + jj, 0), jj=j))
        for j in range(NSTREAMS)
    ]
    out = pl.pallas_call(
        _ffm_kernel,
        grid=grid,
        in_specs=[
            pl.BlockSpec((1, D), lambda i: (0, 0)),
            pl.BlockSpec((1, D), lambda i: (0, 0)),
            *w_specs,
            pl.BlockSpec((1, BLK * NSTREAMS), lambda i: (0, i)),
        ],
        out_specs=pl.BlockSpec((1, BLK * NSTREAMS), lambda i: (0, i)),
        out_shape=jax.ShapeDtypeStruct((1, D_OUT), jnp.float32),
    )(xp, attrs_init, *([W] * NSTREAMS), b2)
    return out


# speedup vs baseline: 1.0048x; 1.0048x over previous
import functools

import jax
import jax.numpy as jnp
from jax.experimental import pallas as pl
from jax.experimental.pallas import tpu as pltpu

D = 8192
D_OUT = 4096
N_FIXED = 8
NSTREAMS = 2
BLK = 256


def _ffm_kernel(xp_ref, attrs_ref, *refs):
    w_refs = refs[:NSTREAMS]
    b_ref, out_ref = refs[NSTREAMS], refs[NSTREAMS + 1]
    xs = pltpu.roll(xp_ref[...], N_FIXED, axis=1)
    col = jax.lax.broadcasted_iota(jnp.int32, (1, D), 1)
    inp = jnp.where(col < N_FIXED, attrs_ref[...], xs)
    for j in range(NSTREAMS):
        acc = jax.lax.dot_general(
            inp, w_refs[j][...], (((1,), (1,)), ((), ())),
            preferred_element_type=jnp.float32)
        out_ref[:, j * BLK:(j + 1) * BLK] = acc + b_ref[:, j * BLK:(j + 1) * BLK]


@functools.partial(jax.jit, static_argnames=())
def kernel(x, attrs_init, W, b):
    xp = jnp.pad(x, ((0, 0), (0, N_FIXED)))
    b2 = b.reshape(1, D_OUT)
    grid = (D_OUT // (BLK * NSTREAMS),)
    w_specs = [
        pl.BlockSpec((BLK, D), functools.partial(
            lambda i, jj: (NSTREAMS * i + jj, 0), jj=j))
        for j in range(NSTREAMS)
    ]
    out = pl.pallas_call(
        _ffm_kernel,
        grid=grid,
        in_specs=[
            pl.BlockSpec((1, D), lambda i: (0, 0)),
            pl.BlockSpec((1, D), lambda i: (0, 0)),
            *w_specs,
            pl.BlockSpec((1, BLK * NSTREAMS), lambda i: (0, i)),
        ],
        out_specs=pl.BlockSpec((1, BLK * NSTREAMS), lambda i: (0, i)),
        out_shape=jax.ShapeDtypeStruct((1, D_OUT), jnp.float32),
    )(xp, attrs_init, *([W] * NSTREAMS), b2)
    return out
